# Initial kernel scaffold; baseline (speedup 1.0000x reference)
import functools

import jax
import jax.numpy as jnp
from jax import lax
from jax.experimental import pallas as pl
from jax.experimental.pallas import tpu as pltpu
from jax.experimental.pallas import tpu_sc as plsc

HID = 128
HEADS = 4
HD = HID // HEADS
T = 3
N = 10000
E = 320000

NC = 2
NS = 16
NW = NC * NS
CHUNK = 128
E_PAD = 327680
EPT = E_PAD // NW
NCHUNK = EPT // CHUNK
RPT = N // NS
AW = 144

BLK = 400
BLKE = 1000

_NEG = -1e30


def _ln(v, g, b):
    u = jnp.mean(v, axis=-1, keepdims=True)
    d = v - u
    var = jnp.mean(d * d, axis=-1, keepdims=True)
    return d / jnp.sqrt(var + 1e-12) * g + b


def _edge_logits_body(ea_ref, ue_ref, out_ref):
    out_ref[...] = jnp.dot(ea_ref[...], ue_ref[...],
                           preferred_element_type=jnp.float32)


def _edge_logits(edge_attr, U_e):
    return pl.pallas_call(
        _edge_logits_body,
        grid=(E // BLKE,),
        in_specs=[pl.BlockSpec((BLKE, HID), lambda i: (i, 0)),
                  pl.BlockSpec((HID, HEADS), lambda i: (0, 0))],
        out_specs=pl.BlockSpec((BLKE, HEADS), lambda i: (i, 0)),
        out_shape=jax.ShapeDtypeStruct((E, HEADS), jnp.float32),
    )(edge_attr, U_e)


def _node_proj_body(x_ref, wn_ref, uij_ref, xn_ref, aij_ref):
    xv = x_ref[...]
    xn_ref[...] = jnp.dot(xv, wn_ref[...], preferred_element_type=jnp.float32)
    aij_ref[...] = jnp.dot(xv, uij_ref[...], preferred_element_type=jnp.float32)


def _node_proj(x, W_node, U_ij):
    return pl.pallas_call(
        _node_proj_body,
        grid=(N // BLK,),
        in_specs=[pl.BlockSpec((BLK, HID), lambda i: (i, 0)),
                  pl.BlockSpec((HID, HID), lambda i: (0, 0)),
                  pl.BlockSpec((HID, 2 * HEADS), lambda i: (0, 0))],
        out_specs=[pl.BlockSpec((BLK, HID), lambda i: (i, 0)),
                   pl.BlockSpec((BLK, 2 * HEADS), lambda i: (i, 0))],
        out_shape=[jax.ShapeDtypeStruct((N, HID), jnp.float32),
                   jax.ShapeDtypeStruct((N, 2 * HEADS), jnp.float32)],
    )(x, W_node, U_ij)


def _dense_body(agg_ref, x_ref, h_ref,
                ws_ref, bs_ref, wao_ref, bao_ref, g1_ref, b1_ref,
                wi_ref, bi_ref, wo_ref, bo_ref, g2_ref, b2_ref,
                wih_ref, whh_ref, bih_ref, bhh_ref, g3_ref, b3_ref,
                wn_ref, uij_ref,
                xo_ref, ho_ref, xno_ref, aijo_ref):
    a = agg_ref[0] + agg_ref[1]
    aggr = a[:, :HID]
    asum = a[:, HID:HID + HEADS]
    recip = 1.0 / (asum + 1e-16)
    lane_head = lax.broadcasted_iota(jnp.int32, (HEADS, HID), 1) // HD
    head_id = lax.broadcasted_iota(jnp.int32, (HEADS, HID), 0)
    sel = (lane_head == head_id).astype(jnp.float32)
    attn_in = aggr * jnp.dot(recip, sel, preferred_element_type=jnp.float32)

    xv = x_ref[...]
    attn = jnp.dot(attn_in, ws_ref[...],
                   preferred_element_type=jnp.float32) + bs_ref[...]
    ao = jnp.dot(attn, wao_ref[...],
                 preferred_element_type=jnp.float32) + bao_ref[...]
    ao = _ln(ao + xv, g1_ref[...], b1_ref[...])

    inter = jnp.dot(ao, wi_ref[...],
                    preferred_element_type=jnp.float32) + bi_ref[...]
    inter = 0.5 * inter * (1.0 + lax.erf(inter * 0.7071067811865476))
    m = jnp.dot(inter, wo_ref[...],
                preferred_element_type=jnp.float32) + bo_ref[...]
    m = _ln(m + ao, g2_ref[...], b2_ref[...])

    hv = h_ref[...]
    gi = jnp.dot(m, wih_ref[...],
                 preferred_element_type=jnp.float32) + bih_ref[...]
    gh = jnp.dot(hv, whh_ref[...],
                 preferred_element_type=jnp.float32) + bhh_ref[...]
    r = jax.nn.sigmoid(gi[:, :HID] + gh[:, :HID])
    z = jax.nn.sigmoid(gi[:, HID:2 * HID] + gh[:, HID:2 * HID])
    ng = jnp.tanh(gi[:, 2 * HID:] + r * gh[:, 2 * HID:])
    hn = (1.0 - z) * ng + z * hv
    xnew = _ln(hn, g3_ref[...], b3_ref[...])

    xo_ref[...] = xnew
    ho_ref[...] = hn
    xno_ref[...] = jnp.dot(xnew, wn_ref[...], preferred_element_type=jnp.float32)
    aijo_ref[...] = jnp.dot(xnew, uij_ref[...], preferred_element_type=jnp.float32)


def _dense(agg2, x, h, wts):
    full = lambda shape: pl.BlockSpec(shape, lambda i: tuple(0 for _ in shape))
    row = lambda w: pl.BlockSpec((BLK, w), lambda i: (i, 0))
    in_specs = [
        pl.BlockSpec((NC, BLK, AW), lambda i: (0, i, 0)),
        row(HID), row(HID),
        full((HID, HID)), full((1, HID)), full((HID, HID)), full((1, HID)),
        full((1, HID)), full((1, HID)),
        full((HID, 4 * HID)), full((1, 4 * HID)),
        full((4 * HID, HID)), full((1, HID)), full((1, HID)), full((1, HID)),
        full((HID, 3 * HID)), full((HID, 3 * HID)),
        full((1, 3 * HID)), full((1, 3 * HID)),
        full((1, HID)), full((1, HID)),
        full((HID, HID)), full((HID, 2 * HEADS)),
    ]
    return pl.pallas_call(
        _dense_body,
        grid=(N // BLK,),
        in_specs=in_specs,
        out_specs=[row(HID), row(HID), row(HID), row(2 * HEADS)],
        out_shape=[jax.ShapeDtypeStruct((N, HID), jnp.float32),
                   jax.ShapeDtypeStruct((N, HID), jnp.float32),
                   jax.ShapeDtypeStruct((N, HID), jnp.float32),
                   jax.ShapeDtypeStruct((N, 2 * HEADS), jnp.float32)],
    )(agg2, x, h, *wts)


_sc_mesh = plsc.VectorSubcoreMesh(core_axis_name="c", subcore_axis_name="s")


@functools.partial(
    pl.kernel,
    out_type=jax.ShapeDtypeStruct((NC, N, AW), jnp.float32),
    mesh=_sc_mesh,
    scratch_types=[
        pltpu.VMEM((N * 2 * HEADS,), jnp.float32),
        pltpu.VMEM((CHUNK,), jnp.int32),
        pltpu.VMEM((CHUNK,), jnp.int32),
        pltpu.VMEM((CHUNK * HEADS,), jnp.float32),
        pltpu.VMEM((CHUNK, HID), jnp.float32),
        pltpu.VMEM((CHUNK, AW), jnp.float32),
        pltpu.VMEM_SHARED((N, AW), jnp.float32),
        pltpu.SemaphoreType.DMA,
    ],
)
def _sc_edge_kernel(src_hbm, dst_hbm, ae_hbm, aij_hbm, xn_hbm, zeros_hbm,
                    out_hbm, aij_v, srci_v, dsti_v, ae_v, rows_v, srow_v,
                    acc_sp, sem):
    c = lax.axis_index("c")
    s = lax.axis_index("s")

    pltpu.sync_copy(aij_hbm, aij_v)
    pltpu.sync_copy(zeros_hbm.at[pl.ds(s * RPT, RPT)],
                    acc_sp.at[pl.ds(s * RPT, RPT)])

    @pl.loop(0, CHUNK)
    def _zero(k):
        srow_v[k, pl.ds(HID, 16)] = jnp.zeros((16,), jnp.float32)

    plsc.subcore_barrier()

    base = (c * NS + s) * EPT

    @pl.loop(0, NCHUNK)
    def _chunk(ci):
        off = base + ci * CHUNK
        pltpu.sync_copy(src_hbm.at[pl.ds(off, CHUNK)], srci_v)
        pltpu.sync_copy(dst_hbm.at[pl.ds(off, CHUNK)], dsti_v)
        pltpu.sync_copy(ae_hbm.at[pl.ds(off * HEADS, CHUNK * HEADS)], ae_v)
        gat = pltpu.async_copy(xn_hbm.at[srci_v], rows_v, sem)

        for g in range(CHUNK // 16):
            rid = lax.iota(jnp.int32, 16) + (g * 16)
            rid4 = rid * HEADS
            dstg = dsti_v[pl.ds(g * 16, 16)] * (2 * HEADS)
            srcg = srci_v[pl.ds(g * 16, 16)] * (2 * HEADS)
            for hh in range(HEADS):
                vi = plsc.load_gather(aij_v, [dstg + hh])
                vj = plsc.load_gather(aij_v, [srcg + (HEADS + hh)])
                ve = plsc.load_gather(ae_v, [rid4 + hh])
                sv = vi + vj + ve
                sv = jnp.maximum(sv, 0.2 * sv)
                wv = jnp.exp(sv)
                plsc.store_scatter(srow_v,
                                   [rid, jnp.full((16,), HID + hh, jnp.int32)],
                                   wv)
        gat.wait()

        @pl.loop(0, CHUNK)
        def _scale(k):
            krep = jnp.broadcast_to(k, (16,))
            for hh in range(HEADS):
                ws = plsc.load_gather(
                    srow_v, [krep, jnp.full((16,), HID + hh, jnp.int32)])
                for q in range(HD // 16):
                    colo = hh * HD + q * 16
                    srow_v[k, pl.ds(colo, 16)] = rows_v[k, pl.ds(colo, 16)] * ws

        pltpu.sync_copy(srow_v, acc_sp.at[dsti_v], add=True)

    plsc.subcore_barrier()
    pltpu.sync_copy(acc_sp.at[pl.ds(s * RPT, RPT)],
                    out_hbm.at[c, pl.ds(s * RPT, RPT)])


def kernel(x, edge_index, edge_attr, W_node, W_edge, att_w, W_scale, b_scale,
           W_ao, b_ao, g_ln1, b_ln1, W_int, b_int, W_out, b_out, g_ln2, b_ln2,
           W_ih, W_hh, b_ih, b_hh, g_ln3, b_ln3):
    src = edge_index[0]
    dst = edge_index[1]

    aw = att_w.reshape(HEADS, 3, HD)
    Wn4 = W_node.reshape(HID, HEADS, HD)
    We4 = W_edge.reshape(HID, HEADS, HD)
    U_i = jnp.einsum('khd,hd->kh', Wn4, aw[:, 0, :])
    U_e = jnp.einsum('khd,hd->kh', We4, aw[:, 1, :])
    U_j = jnp.einsum('khd,hd->kh', Wn4, aw[:, 2, :])
    U_ij = jnp.concatenate([U_i, U_j], axis=1)

    ae = _edge_logits(edge_attr, U_e)
    ae_pad = jnp.concatenate(
        [ae, jnp.full((E_PAD - E, HEADS), _NEG, jnp.float32)]).reshape(-1)
    src_pad = jnp.concatenate([src, jnp.zeros((E_PAD - E,), jnp.int32)])
    dst_pad = jnp.concatenate([dst, jnp.zeros((E_PAD - E,), jnp.int32)])
    zeros = jnp.zeros((N, AW), jnp.float32)

    r2 = lambda v: v.reshape(1, -1)
    wts = (W_scale, r2(b_scale), W_ao, r2(b_ao), r2(g_ln1), r2(b_ln1),
           W_int, r2(b_int), W_out, r2(b_out), r2(g_ln2), r2(b_ln2),
           W_ih.T, W_hh.T, r2(b_ih), r2(b_hh), r2(g_ln3), r2(b_ln3),
           W_node, U_ij)

    xn, aij = _node_proj(x, W_node, U_ij)
    h = x
    for _ in range(T):
        agg2 = _sc_edge_kernel(src_pad, dst_pad, ae_pad, aij.reshape(-1),
                               xn, zeros)
        x, h, xn, aij = _dense(agg2, x, h, wts)
    return x


# speedup vs baseline: 4.2647x; 4.2647x over previous
import dataclasses
import functools

import jax
import jax.numpy as jnp
from jax import lax
from jax.experimental import pallas as pl
from jax.experimental.pallas import tpu as pltpu
from jax.experimental.pallas import tpu_sc as plsc

HID = 128
HEADS = 4
HD = HID // HEADS
T = 3
N = 10000
E = 320000

NC = 2
NS = 16
NW = NC * NS
CHUNK = 128
E_PAD = 327680
EPT = E_PAD // NW
NCHUNK = EPT // CHUNK
N_PAD = 10240
RPT = N_PAD // NS
ASR = N_PAD * HEADS // HID

BLK = 400
BLKE = 1000

_NEG = -1e30


def _ln(v, g, b):
    u = jnp.mean(v, axis=-1, keepdims=True)
    d = v - u
    var = jnp.mean(d * d, axis=-1, keepdims=True)
    return d / jnp.sqrt(var + 1e-12) * g + b


def _edge_logits_body(ea_ref, ue_ref, out_ref):
    out_ref[...] = jnp.dot(ea_ref[...], ue_ref[...],
                           preferred_element_type=jnp.float32)


def _edge_logits(edge_attr, U_e):
    return pl.pallas_call(
        _edge_logits_body,
        grid=(E // BLKE,),
        in_specs=[pl.BlockSpec((BLKE, HID), lambda i: (i, 0)),
                  pl.BlockSpec((HID, HEADS), lambda i: (0, 0))],
        out_specs=pl.BlockSpec((BLKE, HEADS), lambda i: (i, 0)),
        out_shape=jax.ShapeDtypeStruct((E, HEADS), jnp.float32),
    )(edge_attr, U_e)


def _node_proj_body(x_ref, wn_ref, uij_ref, xn_ref, aij_ref):
    xv = x_ref[...]
    xn_ref[...] = jnp.dot(xv, wn_ref[...], preferred_element_type=jnp.float32)
    aij_ref[...] = jnp.dot(xv, uij_ref[...], preferred_element_type=jnp.float32)


def _node_proj(x, W_node, U_ij):
    return pl.pallas_call(
        _node_proj_body,
        grid=(N // BLK,),
        in_specs=[pl.BlockSpec((BLK, HID), lambda i: (i, 0)),
                  pl.BlockSpec((HID, HID), lambda i: (0, 0)),
                  pl.BlockSpec((HID, 2 * HEADS), lambda i: (0, 0))],
        out_specs=[pl.BlockSpec((BLK, HID), lambda i: (i, 0)),
                   pl.BlockSpec((BLK, 2 * HEADS), lambda i: (i, 0))],
        out_shape=[jax.ShapeDtypeStruct((N, HID), jnp.float32),
                   jax.ShapeDtypeStruct((N, 2 * HEADS), jnp.float32)],
    )(x, W_node, U_ij)


def _dense_body(agg_ref, asum_ref, x_ref, h_ref,
                ws_ref, bs_ref, wao_ref, bao_ref, g1_ref, b1_ref,
                wi_ref, bi_ref, wo_ref, bo_ref, g2_ref, b2_ref,
                wih_ref, whh_ref, bih_ref, bhh_ref, g3_ref, b3_ref,
                wn_ref, uij_ref,
                xo_ref, ho_ref, xno_ref, aijo_ref):
    aggr = agg_ref[0] + agg_ref[1]
    asum = asum_ref[0] + asum_ref[1]
    recip = 1.0 / (asum + 1e-16)
    lane_head = lax.broadcasted_iota(jnp.int32, (HEADS, HID), 1) // HD
    head_id = lax.broadcasted_iota(jnp.int32, (HEADS, HID), 0)
    sel = (lane_head == head_id).astype(jnp.float32)
    attn_in = aggr * jnp.dot(recip, sel, preferred_element_type=jnp.float32)

    xv = x_ref[...]
    attn = jnp.dot(attn_in, ws_ref[...],
                   preferred_element_type=jnp.float32) + bs_ref[...]
    ao = jnp.dot(attn, wao_ref[...],
                 preferred_element_type=jnp.float32) + bao_ref[...]
    ao = _ln(ao + xv, g1_ref[...], b1_ref[...])

    inter = jnp.dot(ao, wi_ref[...],
                    preferred_element_type=jnp.float32) + bi_ref[...]
    inter = 0.5 * inter * (1.0 + lax.erf(inter * 0.7071067811865476))
    m = jnp.dot(inter, wo_ref[...],
                preferred_element_type=jnp.float32) + bo_ref[...]
    m = _ln(m + ao, g2_ref[...], b2_ref[...])

    hv = h_ref[...]
    gi = jnp.dot(m, wih_ref[...],
                 preferred_element_type=jnp.float32) + bih_ref[...]
    gh = jnp.dot(hv, whh_ref[...],
                 preferred_element_type=jnp.float32) + bhh_ref[...]
    r = jax.nn.sigmoid(gi[:, :HID] + gh[:, :HID])
    z = jax.nn.sigmoid(gi[:, HID:2 * HID] + gh[:, HID:2 * HID])
    ng = jnp.tanh(gi[:, 2 * HID:] + r * gh[:, 2 * HID:])
    hn = (1.0 - z) * ng + z * hv
    xnew = _ln(hn, g3_ref[...], b3_ref[...])

    xo_ref[...] = xnew
    ho_ref[...] = hn
    xno_ref[...] = jnp.dot(xnew, wn_ref[...], preferred_element_type=jnp.float32)
    aijo_ref[...] = jnp.dot(xnew, uij_ref[...], preferred_element_type=jnp.float32)


def _dense(agg2, asum4, x, h, wts):
    full = lambda shape: pl.BlockSpec(shape, lambda i: tuple(0 for _ in shape))
    row = lambda w: pl.BlockSpec((BLK, w), lambda i: (i, 0))
    in_specs = [
        pl.BlockSpec((NC, BLK, HID), lambda i: (0, i, 0)),
        pl.BlockSpec((NC, BLK, HEADS), lambda i: (0, i, 0)),
        row(HID), row(HID),
        full((HID, HID)), full((1, HID)), full((HID, HID)), full((1, HID)),
        full((1, HID)), full((1, HID)),
        full((HID, 4 * HID)), full((1, 4 * HID)),
        full((4 * HID, HID)), full((1, HID)), full((1, HID)), full((1, HID)),
        full((HID, 3 * HID)), full((HID, 3 * HID)),
        full((1, 3 * HID)), full((1, 3 * HID)),
        full((1, HID)), full((1, HID)),
        full((HID, HID)), full((HID, 2 * HEADS)),
    ]
    return pl.pallas_call(
        _dense_body,
        grid=(N // BLK,),
        in_specs=in_specs,
        out_specs=[row(HID), row(HID), row(HID), row(2 * HEADS)],
        out_shape=[jax.ShapeDtypeStruct((N, HID), jnp.float32),
                   jax.ShapeDtypeStruct((N, HID), jnp.float32),
                   jax.ShapeDtypeStruct((N, HID), jnp.float32),
                   jax.ShapeDtypeStruct((N, 2 * HEADS), jnp.float32)],
    )(agg2, asum4, x, h, *wts)


def _sc_weights_body(src_hbm, dst_hbm, ae_hbm, aij_hbm, iota_hbm, zeros_hbm,
                     w_hbm, asum_hbm, aij_v, srci_v, dsti_v, ae_v, w_v,
                     iota_v, asum_loc, asum_sp, sem):
    c = lax.axis_index("c")
    s = lax.axis_index("s")

    pltpu.sync_copy(aij_hbm, aij_v)
    pltpu.sync_copy(iota_hbm, iota_v)
    pltpu.sync_copy(zeros_hbm.at[pl.ds(0, ASR)], asum_loc)

    @pl.when(s < 10)
    def _():
        pltpu.sync_copy(zeros_hbm.at[pl.ds(0, 32)],
                        asum_sp.at[pl.ds(s * 32, 32)])

    plsc.subcore_barrier()

    base = (c * NS + s) * EPT

    @pl.loop(0, NCHUNK)
    def _chunk(ci):
        off = base + ci * CHUNK
        pltpu.sync_copy(src_hbm.at[pl.ds(off, CHUNK)], srci_v)
        pltpu.sync_copy(dst_hbm.at[pl.ds(off, CHUNK)], dsti_v)
        pltpu.sync_copy(ae_hbm.at[pl.ds(off * HEADS, CHUNK * HEADS)], ae_v)

        for g in range(CHUNK // 16):
            rid4 = (lax.iota(jnp.int32, 16) + g * 16) * HEADS
            dstg = dsti_v[pl.ds(g * 16, 16)]
            srcg = srci_v[pl.ds(g * 16, 16)]
            d8 = dstg * (2 * HEADS)
            s8 = srcg * (2 * HEADS) + HEADS
            d4 = dstg * HEADS
            for hh in range(HEADS):
                vi = plsc.load_gather(aij_v, [d8 + hh])
                vj = plsc.load_gather(aij_v, [s8 + hh])
                ve = plsc.load_gather(ae_v, [rid4 + hh])
                sv = vi + vj + ve
                sv = jnp.maximum(sv, 0.2 * sv)
                wv = jnp.exp(sv)
                plsc.store_scatter(w_v, [rid4 + hh], wv)
                f = d4 + hh
                plsc.addupdate_scatter(
                    asum_loc,
                    [lax.shift_right_logical(f, 7), lax.bitwise_and(f, 127)],
                    wv)

        pltpu.sync_copy(w_v, w_hbm.at[pl.ds(off * HEADS, CHUNK * HEADS)])

    for j in range(HEADS):
        pltpu.sync_copy(asum_loc.at[pl.ds(j * (ASR // HEADS), ASR // HEADS)],
                        asum_sp.at[iota_v.at[j]], add=True)

    plsc.subcore_barrier()

    @pl.when(s < 10)
    def _():
        pltpu.sync_copy(asum_sp.at[pl.ds(s * 32, 32)],
                        asum_hbm.at[c, pl.ds(s * 32, 32)])


def _sc_aggr_body(src_hbm, dst_hbm, w_hbm, xn_hbm, zeros_hbm,
                  agg_hbm, srci_v, dsti_v, w_v, rows_v, acc_sp, sem):
    c = lax.axis_index("c")
    s = lax.axis_index("s")

    pltpu.sync_copy(zeros_hbm.at[pl.ds(0, RPT)], acc_sp.at[pl.ds(s * RPT, RPT)])
    plsc.subcore_barrier()

    base = (c * NS + s) * EPT

    @pl.loop(0, NCHUNK)
    def _chunk(ci):
        off = base + ci * CHUNK
        pltpu.sync_copy(src_hbm.at[pl.ds(off, CHUNK)], srci_v)
        pltpu.sync_copy(dst_hbm.at[pl.ds(off, CHUNK)], dsti_v)
        pltpu.sync_copy(w_hbm.at[pl.ds(off * HEADS, CHUNK * HEADS)], w_v)
        pltpu.async_copy(xn_hbm.at[srci_v], rows_v, sem).wait()

        @pl.loop(0, CHUNK)
        def _scale(k):
            k4 = k * HEADS
            for hh in range(HEADS):
                ws = plsc.load_gather(w_v, [jnp.broadcast_to(k4 + hh, (16,))])
                for q in range(HD // 16):
                    colo = hh * HD + q * 16
                    rows_v[k, pl.ds(colo, 16)] = rows_v[k, pl.ds(colo, 16)] * ws

        pltpu.sync_copy(rows_v, acc_sp.at[dsti_v], add=True)

    plsc.subcore_barrier()
    pltpu.sync_copy(acc_sp.at[pl.ds(s * RPT, RPT)],
                    agg_hbm.at[c, pl.ds(s * RPT, RPT)])


def _sc_compiler_params():
    cp = pltpu.CompilerParams()
    if "needs_layout_passes" in pltpu.CompilerParams.__dataclass_fields__:
        cp = dataclasses.replace(cp, needs_layout_passes=False)
    return cp


@functools.cache
def _sc_weights_kernel():
    mesh = plsc.VectorSubcoreMesh(core_axis_name="c", subcore_axis_name="s")
    return pl.kernel(
        _sc_weights_body,
        out_type=[jax.ShapeDtypeStruct((E_PAD * HEADS,), jnp.float32),
                  jax.ShapeDtypeStruct((NC, ASR, HID), jnp.float32)],
        mesh=mesh,
        compiler_params=_sc_compiler_params(),
        scratch_types=[
            pltpu.VMEM((N * 2 * HEADS,), jnp.float32),
            pltpu.VMEM((CHUNK,), jnp.int32),
            pltpu.VMEM((CHUNK,), jnp.int32),
            pltpu.VMEM((CHUNK * HEADS,), jnp.float32),
            pltpu.VMEM((CHUNK * HEADS,), jnp.float32),
            pltpu.VMEM((HEADS, ASR // HEADS), jnp.int32),
            pltpu.VMEM((ASR, HID), jnp.float32),
            pltpu.VMEM_SHARED((ASR, HID), jnp.float32),
            pltpu.SemaphoreType.DMA,
        ],
    )


@functools.cache
def _sc_aggr_kernel():
    mesh = plsc.VectorSubcoreMesh(core_axis_name="c", subcore_axis_name="s")
    return pl.kernel(
        _sc_aggr_body,
        out_type=jax.ShapeDtypeStruct((NC, N_PAD, HID), jnp.float32),
        mesh=mesh,
        compiler_params=_sc_compiler_params(),
        scratch_types=[
            pltpu.VMEM((CHUNK,), jnp.int32),
            pltpu.VMEM((CHUNK,), jnp.int32),
            pltpu.VMEM((CHUNK * HEADS,), jnp.float32),
            pltpu.VMEM((CHUNK, HID), jnp.float32),
            pltpu.VMEM_SHARED((N_PAD, HID), jnp.float32),
            pltpu.SemaphoreType.DMA,
        ],
    )


def kernel(x, edge_index, edge_attr, W_node, W_edge, att_w, W_scale, b_scale,
           W_ao, b_ao, g_ln1, b_ln1, W_int, b_int, W_out, b_out, g_ln2, b_ln2,
           W_ih, W_hh, b_ih, b_hh, g_ln3, b_ln3):
    src = edge_index[0]
    dst = edge_index[1]

    aw = att_w.reshape(HEADS, 3, HD)
    Wn4 = W_node.reshape(HID, HEADS, HD)
    We4 = W_edge.reshape(HID, HEADS, HD)
    U_i = jnp.einsum('khd,hd->kh', Wn4, aw[:, 0, :])
    U_e = jnp.einsum('khd,hd->kh', We4, aw[:, 1, :])
    U_j = jnp.einsum('khd,hd->kh', Wn4, aw[:, 2, :])
    U_ij = jnp.concatenate([U_i, U_j], axis=1)

    ae = _edge_logits(edge_attr, U_e)
    ae_pad = jnp.concatenate(
        [ae, jnp.full((E_PAD - E, HEADS), _NEG, jnp.float32)]).reshape(-1)
    src_pad = jnp.concatenate([src, jnp.zeros((E_PAD - E,), jnp.int32)])
    dst_pad = jnp.concatenate([dst, jnp.zeros((E_PAD - E,), jnp.int32)])
    zeros = jnp.zeros((N_PAD, HID), jnp.float32)
    iota = jnp.arange(ASR, dtype=jnp.int32).reshape(HEADS, ASR // HEADS)

    r2 = lambda v: v.reshape(1, -1)
    wts = (W_scale, r2(b_scale), W_ao, r2(b_ao), r2(g_ln1), r2(b_ln1),
           W_int, r2(b_int), W_out, r2(b_out), r2(g_ln2), r2(b_ln2),
           W_ih.T, W_hh.T, r2(b_ih), r2(b_hh), r2(g_ln3), r2(b_ln3),
           W_node, U_ij)

    xn, aij = _node_proj(x, W_node, U_ij)
    h = x
    for _ in range(T):
        w_e, asum2 = _sc_weights_kernel()(src_pad, dst_pad, ae_pad,
                                          aij.reshape(-1), iota, zeros)
        agg2 = _sc_aggr_kernel()(src_pad, dst_pad, w_e, xn, zeros)
        asum4 = asum2.reshape(NC, N_PAD, HEADS)
        x, h, xn, aij = _dense(agg2, asum4, x, h, wts)
    return x


# speedup vs baseline: 5.9470x; 1.3945x over previous
import dataclasses
import functools

import jax
import jax.numpy as jnp
from jax import lax
from jax.experimental import pallas as pl
from jax.experimental.pallas import tpu as pltpu
from jax.experimental.pallas import tpu_sc as plsc

HID = 128
HEADS = 4
HD = HID // HEADS
T = 3
N = 10000
E = 320000

NC = 2
NS = 16
NW = NC * NS
CHUNK = 128
E_PAD = 327680
EPT = E_PAD // NW
NCHUNK = EPT // CHUNK
N_PAD = 10240
RPT = N_PAD // NS
ASR = N_PAD * HEADS // HID

BLK = 400
BLKE = 1000

_NEG = -1e30


def _ln(v, g, b):
    u = jnp.mean(v, axis=-1, keepdims=True)
    d = v - u
    var = jnp.mean(d * d, axis=-1, keepdims=True)
    return d / jnp.sqrt(var + 1e-12) * g + b


def _edge_logits_body(ea_ref, ue_ref, out_ref):
    out_ref[...] = jnp.dot(ea_ref[...], ue_ref[...],
                           preferred_element_type=jnp.float32)


def _edge_logits(edge_attr, U_e):
    return pl.pallas_call(
        _edge_logits_body,
        grid=(E // BLKE,),
        in_specs=[pl.BlockSpec((BLKE, HID), lambda i: (i, 0)),
                  pl.BlockSpec((HID, HEADS), lambda i: (0, 0))],
        out_specs=pl.BlockSpec((BLKE, HEADS), lambda i: (i, 0)),
        out_shape=jax.ShapeDtypeStruct((E, HEADS), jnp.float32),
    )(edge_attr, U_e)


def _node_proj_body(x_ref, wn_ref, uij_ref, xn_ref, aij_ref):
    xv = x_ref[...]
    xn_ref[...] = jnp.dot(xv, wn_ref[...], preferred_element_type=jnp.float32)
    aij_ref[...] = jnp.dot(xv, uij_ref[...], preferred_element_type=jnp.float32)


def _node_proj(x, W_node, U_ij):
    return pl.pallas_call(
        _node_proj_body,
        grid=(N // BLK,),
        in_specs=[pl.BlockSpec((BLK, HID), lambda i: (i, 0)),
                  pl.BlockSpec((HID, HID), lambda i: (0, 0)),
                  pl.BlockSpec((HID, 2 * HEADS), lambda i: (0, 0))],
        out_specs=[pl.BlockSpec((BLK, HID), lambda i: (i, 0)),
                   pl.BlockSpec((BLK, 2 * HEADS), lambda i: (i, 0))],
        out_shape=[jax.ShapeDtypeStruct((N, HID), jnp.float32),
                   jax.ShapeDtypeStruct((N, 2 * HEADS), jnp.float32)],
    )(x, W_node, U_ij)


def _dense_body(agg_ref, asum_ref, x_ref, h_ref,
                ws_ref, bs_ref, wao_ref, bao_ref, g1_ref, b1_ref,
                wi_ref, bi_ref, wo_ref, bo_ref, g2_ref, b2_ref,
                wih_ref, whh_ref, bih_ref, bhh_ref, g3_ref, b3_ref,
                wn_ref, uij_ref,
                xo_ref, ho_ref, xno_ref, aijo_ref):
    aggr = agg_ref[0] + agg_ref[1]
    asum = asum_ref[0] + asum_ref[1]
    recip = 1.0 / (asum + 1e-16)
    lane_head = lax.broadcasted_iota(jnp.int32, (HEADS, HID), 1) // HD
    head_id = lax.broadcasted_iota(jnp.int32, (HEADS, HID), 0)
    sel = (lane_head == head_id).astype(jnp.float32)
    attn_in = aggr * jnp.dot(recip, sel, preferred_element_type=jnp.float32)

    xv = x_ref[...]
    attn = jnp.dot(attn_in, ws_ref[...],
                   preferred_element_type=jnp.float32) + bs_ref[...]
    ao = jnp.dot(attn, wao_ref[...],
                 preferred_element_type=jnp.float32) + bao_ref[...]
    ao = _ln(ao + xv, g1_ref[...], b1_ref[...])

    inter = jnp.dot(ao, wi_ref[...],
                    preferred_element_type=jnp.float32) + bi_ref[...]
    inter = 0.5 * inter * (1.0 + lax.erf(inter * 0.7071067811865476))
    m = jnp.dot(inter, wo_ref[...],
                preferred_element_type=jnp.float32) + bo_ref[...]
    m = _ln(m + ao, g2_ref[...], b2_ref[...])

    hv = h_ref[...]
    gi = jnp.dot(m, wih_ref[...],
                 preferred_element_type=jnp.float32) + bih_ref[...]
    gh = jnp.dot(hv, whh_ref[...],
                 preferred_element_type=jnp.float32) + bhh_ref[...]
    r = jax.nn.sigmoid(gi[:, :HID] + gh[:, :HID])
    z = jax.nn.sigmoid(gi[:, HID:2 * HID] + gh[:, HID:2 * HID])
    ng = jnp.tanh(gi[:, 2 * HID:] + r * gh[:, 2 * HID:])
    hn = (1.0 - z) * ng + z * hv
    xnew = _ln(hn, g3_ref[...], b3_ref[...])

    xo_ref[...] = xnew
    ho_ref[...] = hn
    xno_ref[...] = jnp.dot(xnew, wn_ref[...], preferred_element_type=jnp.float32)
    aijo_ref[...] = jnp.dot(xnew, uij_ref[...], preferred_element_type=jnp.float32)


def _dense(agg2, asum4, x, h, wts):
    full = lambda shape: pl.BlockSpec(shape, lambda i: tuple(0 for _ in shape))
    row = lambda w: pl.BlockSpec((BLK, w), lambda i: (i, 0))
    in_specs = [
        pl.BlockSpec((NC, BLK, HID), lambda i: (0, i, 0)),
        pl.BlockSpec((NC, BLK, HEADS), lambda i: (0, i, 0)),
        row(HID), row(HID),
        full((HID, HID)), full((1, HID)), full((HID, HID)), full((1, HID)),
        full((1, HID)), full((1, HID)),
        full((HID, 4 * HID)), full((1, 4 * HID)),
        full((4 * HID, HID)), full((1, HID)), full((1, HID)), full((1, HID)),
        full((HID, 3 * HID)), full((HID, 3 * HID)),
        full((1, 3 * HID)), full((1, 3 * HID)),
        full((1, HID)), full((1, HID)),
        full((HID, HID)), full((HID, 2 * HEADS)),
    ]
    return pl.pallas_call(
        _dense_body,
        grid=(N // BLK,),
        in_specs=in_specs,
        out_specs=[row(HID), row(HID), row(HID), row(2 * HEADS)],
        out_shape=[jax.ShapeDtypeStruct((N, HID), jnp.float32),
                   jax.ShapeDtypeStruct((N, HID), jnp.float32),
                   jax.ShapeDtypeStruct((N, HID), jnp.float32),
                   jax.ShapeDtypeStruct((N, 2 * HEADS), jnp.float32)],
    )(agg2, asum4, x, h, *wts)


def _sc_weights_body(src_hbm, dst_hbm, ae_hbm, aij_hbm, iota_hbm, zeros_hbm,
                     w_hbm, asum_hbm, aij_v, srci_v, dsti_v, ae_v, w_v,
                     iota_v, asum_loc, asum_sp, sem):
    c = lax.axis_index("c")
    s = lax.axis_index("s")

    pltpu.sync_copy(aij_hbm, aij_v)
    pltpu.sync_copy(iota_hbm, iota_v)
    pltpu.sync_copy(zeros_hbm.at[pl.ds(0, ASR)], asum_loc)

    @pl.when(s < 10)
    def _():
        pltpu.sync_copy(zeros_hbm.at[pl.ds(0, 32)],
                        asum_sp.at[pl.ds(s * 32, 32)])

    plsc.subcore_barrier()

    base = (c * NS + s) * EPT

    @pl.loop(0, NCHUNK)
    def _chunk(ci):
        off = base + ci * CHUNK
        pltpu.sync_copy(src_hbm.at[pl.ds(off, CHUNK)], srci_v)
        pltpu.sync_copy(dst_hbm.at[pl.ds(off, CHUNK)], dsti_v)
        pltpu.sync_copy(ae_hbm.at[pl.ds(off * HEADS, CHUNK * HEADS)], ae_v)

        for g in range(CHUNK // 16):
            rid4 = (lax.iota(jnp.int32, 16) + g * 16) * HEADS
            dstg = dsti_v[pl.ds(g * 16, 16)]
            srcg = srci_v[pl.ds(g * 16, 16)]
            d8 = dstg * (2 * HEADS)
            s8 = srcg * (2 * HEADS) + HEADS
            d4 = dstg * HEADS
            for hh in range(HEADS):
                vi = plsc.load_gather(aij_v, [d8 + hh])
                vj = plsc.load_gather(aij_v, [s8 + hh])
                ve = plsc.load_gather(ae_v, [rid4 + hh])
                sv = vi + vj + ve
                sv = jnp.maximum(sv, 0.2 * sv)
                wv = jnp.exp(sv)
                plsc.store_scatter(w_v, [rid4 + hh], wv)
                f = d4 + hh
                plsc.addupdate_scatter(
                    asum_loc,
                    [lax.shift_right_logical(f, 7), lax.bitwise_and(f, 127)],
                    wv)

        pltpu.sync_copy(w_v, w_hbm.at[pl.ds(off * HEADS, CHUNK * HEADS)])

    for j in range(HEADS):
        pltpu.sync_copy(asum_loc.at[pl.ds(j * (ASR // HEADS), ASR // HEADS)],
                        asum_sp.at[iota_v.at[j]], add=True)

    plsc.subcore_barrier()

    @pl.when(s < 10)
    def _():
        pltpu.sync_copy(asum_sp.at[pl.ds(s * 32, 32)],
                        asum_hbm.at[c, pl.ds(s * 32, 32)])


CB = 64
NCB = EPT // CB
NSLOT = 5


def _sc_aggr_body(src_hbm, dst_hbm, w_hbm, xn_hbm, zeros_hbm,
                  agg_hbm, *scr):
    srci = scr[0:NSLOT]
    dsti = scr[NSLOT:2 * NSLOT]
    wv = scr[2 * NSLOT:3 * NSLOT]
    rows = scr[3 * NSLOT:4 * NSLOT]
    acc_sp = scr[4 * NSLOT]
    sidx = scr[4 * NSLOT + 1:4 * NSLOT + 1 + NSLOT]
    sgat = scr[4 * NSLOT + 1 + NSLOT:4 * NSLOT + 1 + 2 * NSLOT]
    ssc = scr[4 * NSLOT + 1 + 2 * NSLOT:4 * NSLOT + 1 + 3 * NSLOT]

    c = lax.axis_index("c")
    s = lax.axis_index("s")
    base_e = (c * NS + s) * EPT

    pltpu.sync_copy(zeros_hbm.at[pl.ds(0, RPT)], acc_sp.at[pl.ds(s * RPT, RPT)])
    plsc.subcore_barrier()

    def fire_idx(i, u):
        e = base_e + i * CB
        pltpu.async_copy(src_hbm.at[pl.ds(e, CB)], srci[u], sidx[u])
        pltpu.async_copy(dst_hbm.at[pl.ds(e, CB)], dsti[u], sidx[u])
        pltpu.async_copy(w_hbm.at[pl.ds(e * HEADS, CB * HEADS)], wv[u], sidx[u])

    def wait_idx(u):
        pltpu.make_async_copy(src_hbm.at[pl.ds(0, CB)], srci[u], sidx[u]).wait()
        pltpu.make_async_copy(dst_hbm.at[pl.ds(0, CB)], dsti[u], sidx[u]).wait()
        pltpu.make_async_copy(w_hbm.at[pl.ds(0, CB * HEADS)], wv[u],
                              sidx[u]).wait()

    def fire_gather(u):
        pltpu.async_copy(xn_hbm.at[srci[u]], rows[u], sgat[u])

    def wait_gather(u):
        pltpu.make_async_copy(xn_hbm.at[pl.ds(0, CB)], rows[u], sgat[u]).wait()

    def fire_scatter(u):
        return pltpu.async_copy(rows[u], acc_sp.at[dsti[u]], ssc[u], add=True)

    def wait_scatter(u):
        pltpu.make_async_copy(xn_hbm.at[pl.ds(0, CB)], rows[u], ssc[u]).wait()

    def scale(u):
        ru = rows[u]
        wu = wv[u]

        @pl.loop(0, CB)
        def _(k):
            k4 = k * HEADS
            for hh in range(HEADS):
                ws = plsc.load_gather(wu, [jnp.broadcast_to(k4 + hh, (16,))])
                for q in range(HD // 16):
                    colo = hh * HD + q * 16
                    ru[k, pl.ds(colo, 16)] = ru[k, pl.ds(colo, 16)] * ws

    def sub(i, u, skip_ssc=False, pf_idx=True, pf_gat=True):
        wait_gather(u)
        scale(u)
        h = fire_scatter(u)
        if pf_idx:
            v3 = (u + 3) % NSLOT
            if not skip_ssc:
                wait_scatter(v3)
            fire_idx(i + 3, v3)
        if pf_gat:
            v2 = (u + 2) % NSLOT
            wait_idx(v2)
            fire_gather(v2)
        return h

    fire_idx(0, 0)
    fire_idx(1, 1)
    fire_idx(2, 2)
    wait_idx(0)
    fire_gather(0)
    wait_idx(1)
    fire_gather(1)
    sub(0, 0, skip_ssc=True)
    sub(1, 1, skip_ssc=True)
    sub(2, 2)
    sub(3, 3)
    sub(4, 4)

    @pl.loop(1, NCB // NSLOT - 1)
    def _(j):
        i0 = j * NSLOT
        for u in range(NSLOT):
            sub(i0 + u, u)

    i0 = NCB - NSLOT
    hs = [sub(i0, 0),
          sub(i0 + 1, 1),
          sub(i0 + 2, 2, pf_idx=False),
          sub(i0 + 3, 3, pf_idx=False, pf_gat=False),
          sub(i0 + 4, 4, pf_idx=False, pf_gat=False)]
    for h in hs:
        h.wait()

    plsc.subcore_barrier()
    pltpu.sync_copy(acc_sp.at[pl.ds(s * RPT, RPT)],
                    agg_hbm.at[c, pl.ds(s * RPT, RPT)])


def _sc_compiler_params():
    cp = pltpu.CompilerParams()
    if "needs_layout_passes" in pltpu.CompilerParams.__dataclass_fields__:
        cp = dataclasses.replace(cp, needs_layout_passes=False)
    return cp


@functools.cache
def _sc_weights_kernel():
    mesh = plsc.VectorSubcoreMesh(core_axis_name="c", subcore_axis_name="s")
    return pl.kernel(
        _sc_weights_body,
        out_type=[jax.ShapeDtypeStruct((E_PAD * HEADS,), jnp.float32),
                  jax.ShapeDtypeStruct((NC, ASR, HID), jnp.float32)],
        mesh=mesh,
        compiler_params=_sc_compiler_params(),
        scratch_types=[
            pltpu.VMEM((N * 2 * HEADS,), jnp.float32),
            pltpu.VMEM((CHUNK,), jnp.int32),
            pltpu.VMEM((CHUNK,), jnp.int32),
            pltpu.VMEM((CHUNK * HEADS,), jnp.float32),
            pltpu.VMEM((CHUNK * HEADS,), jnp.float32),
            pltpu.VMEM((HEADS, ASR // HEADS), jnp.int32),
            pltpu.VMEM((ASR, HID), jnp.float32),
            pltpu.VMEM_SHARED((ASR, HID), jnp.float32),
            pltpu.SemaphoreType.DMA,
        ],
    )


@functools.cache
def _sc_aggr_kernel():
    mesh = plsc.VectorSubcoreMesh(core_axis_name="c", subcore_axis_name="s")
    return pl.kernel(
        _sc_aggr_body,
        out_type=jax.ShapeDtypeStruct((NC, N_PAD, HID), jnp.float32),
        mesh=mesh,
        compiler_params=_sc_compiler_params(),
        scratch_types=(
            [pltpu.VMEM((CB,), jnp.int32)] * NSLOT
            + [pltpu.VMEM((CB,), jnp.int32)] * NSLOT
            + [pltpu.VMEM((CB * HEADS,), jnp.float32)] * NSLOT
            + [pltpu.VMEM((CB, HID), jnp.float32)] * NSLOT
            + [pltpu.VMEM_SHARED((N_PAD, HID), jnp.float32)]
            + [pltpu.SemaphoreType.DMA] * (3 * NSLOT)
        ),
    )


def kernel(x, edge_index, edge_attr, W_node, W_edge, att_w, W_scale, b_scale,
           W_ao, b_ao, g_ln1, b_ln1, W_int, b_int, W_out, b_out, g_ln2, b_ln2,
           W_ih, W_hh, b_ih, b_hh, g_ln3, b_ln3):
    src = edge_index[0]
    dst = edge_index[1]

    aw = att_w.reshape(HEADS, 3, HD)
    Wn4 = W_node.reshape(HID, HEADS, HD)
    We4 = W_edge.reshape(HID, HEADS, HD)
    U_i = jnp.einsum('khd,hd->kh', Wn4, aw[:, 0, :])
    U_e = jnp.einsum('khd,hd->kh', We4, aw[:, 1, :])
    U_j = jnp.einsum('khd,hd->kh', Wn4, aw[:, 2, :])
    U_ij = jnp.concatenate([U_i, U_j], axis=1)

    ae = _edge_logits(edge_attr, U_e)
    ae_pad = jnp.concatenate(
        [ae, jnp.full((E_PAD - E, HEADS), _NEG, jnp.float32)]).reshape(-1)
    src_pad = jnp.concatenate([src, jnp.zeros((E_PAD - E,), jnp.int32)])
    dst_pad = jnp.concatenate([dst, jnp.zeros((E_PAD - E,), jnp.int32)])
    zeros = jnp.zeros((N_PAD, HID), jnp.float32)
    iota = jnp.arange(ASR, dtype=jnp.int32).reshape(HEADS, ASR // HEADS)

    r2 = lambda v: v.reshape(1, -1)
    wts = (W_scale, r2(b_scale), W_ao, r2(b_ao), r2(g_ln1), r2(b_ln1),
           W_int, r2(b_int), W_out, r2(b_out), r2(g_ln2), r2(b_ln2),
           W_ih.T, W_hh.T, r2(b_ih), r2(b_hh), r2(g_ln3), r2(b_ln3),
           W_node, U_ij)

    xn, aij = _node_proj(x, W_node, U_ij)
    h = x
    for _ in range(T):
        w_e, asum2 = _sc_weights_kernel()(src_pad, dst_pad, ae_pad,
                                          aij.reshape(-1), iota, zeros)
        agg2 = _sc_aggr_kernel()(src_pad, dst_pad, w_e, xn, zeros)
        asum4 = asum2.reshape(NC, N_PAD, HEADS)
        x, h, xn, aij = _dense(agg2, asum4, x, h, wts)
    return x


# speedup vs baseline: 6.0984x; 1.0255x over previous
import dataclasses
import functools

import jax
import jax.numpy as jnp
from jax import lax
from jax.experimental import pallas as pl
from jax.experimental.pallas import tpu as pltpu
from jax.experimental.pallas import tpu_sc as plsc

HID = 128
HEADS = 4
HD = HID // HEADS
T = 3
N = 10000
E = 320000

NC = 2
NS = 16
NW = NC * NS
CHUNK = 128
E_PAD = 327680
EPT = E_PAD // NW
NCHUNK = EPT // CHUNK
N_PAD = 10240
RPT = N_PAD // NS
ASR = N_PAD * HEADS // HID

BLK = 400
BLKE = 1000

_NEG = -1e30


def _ln(v, g, b):
    u = jnp.mean(v, axis=-1, keepdims=True)
    d = v - u
    var = jnp.mean(d * d, axis=-1, keepdims=True)
    return d / jnp.sqrt(var + 1e-12) * g + b


def _edge_logits_body(ea_ref, ue_ref, out_ref):
    out_ref[...] = jnp.dot(ea_ref[...], ue_ref[...],
                           preferred_element_type=jnp.float32)


def _edge_logits(edge_attr, U_e):
    return pl.pallas_call(
        _edge_logits_body,
        grid=(E // BLKE,),
        in_specs=[pl.BlockSpec((BLKE, HID), lambda i: (i, 0)),
                  pl.BlockSpec((HID, HEADS), lambda i: (0, 0))],
        out_specs=pl.BlockSpec((BLKE, HEADS), lambda i: (i, 0)),
        out_shape=jax.ShapeDtypeStruct((E, HEADS), jnp.float32),
    )(edge_attr, U_e)


def _node_proj_body(x_ref, wn_ref, uij_ref, xn_ref, aij_ref):
    xv = x_ref[...]
    xn_ref[...] = jnp.dot(xv, wn_ref[...], preferred_element_type=jnp.float32)
    aij_ref[...] = jnp.dot(xv, uij_ref[...], preferred_element_type=jnp.float32)


def _node_proj(x, W_node, U_ij):
    return pl.pallas_call(
        _node_proj_body,
        grid=(N // BLK,),
        in_specs=[pl.BlockSpec((BLK, HID), lambda i: (i, 0)),
                  pl.BlockSpec((HID, HID), lambda i: (0, 0)),
                  pl.BlockSpec((HID, 2 * HEADS), lambda i: (0, 0))],
        out_specs=[pl.BlockSpec((BLK, HID), lambda i: (i, 0)),
                   pl.BlockSpec((BLK, 2 * HEADS), lambda i: (i, 0))],
        out_shape=[jax.ShapeDtypeStruct((N, HID), jnp.float32),
                   jax.ShapeDtypeStruct((N, 2 * HEADS), jnp.float32)],
    )(x, W_node, U_ij)


def _dense_body(agg_ref, asum_ref, x_ref, h_ref,
                ws_ref, bs_ref, wao_ref, bao_ref, g1_ref, b1_ref,
                wi_ref, bi_ref, wo_ref, bo_ref, g2_ref, b2_ref,
                wih_ref, whh_ref, bih_ref, bhh_ref, g3_ref, b3_ref,
                wn_ref, uij_ref,
                xo_ref, ho_ref, xno_ref, aijo_ref):
    aggr = agg_ref[0] + agg_ref[1]
    asum = asum_ref[0] + asum_ref[1]
    recip = 1.0 / (asum + 1e-16)
    lane_head = lax.broadcasted_iota(jnp.int32, (HEADS, HID), 1) // HD
    head_id = lax.broadcasted_iota(jnp.int32, (HEADS, HID), 0)
    sel = (lane_head == head_id).astype(jnp.float32)
    attn_in = aggr * jnp.dot(recip, sel, preferred_element_type=jnp.float32)

    xv = x_ref[...]
    attn = jnp.dot(attn_in, ws_ref[...],
                   preferred_element_type=jnp.float32) + bs_ref[...]
    ao = jnp.dot(attn, wao_ref[...],
                 preferred_element_type=jnp.float32) + bao_ref[...]
    ao = _ln(ao + xv, g1_ref[...], b1_ref[...])

    inter = jnp.dot(ao, wi_ref[...],
                    preferred_element_type=jnp.float32) + bi_ref[...]
    inter = 0.5 * inter * (1.0 + lax.erf(inter * 0.7071067811865476))
    m = jnp.dot(inter, wo_ref[...],
                preferred_element_type=jnp.float32) + bo_ref[...]
    m = _ln(m + ao, g2_ref[...], b2_ref[...])

    hv = h_ref[...]
    gi = jnp.dot(m, wih_ref[...],
                 preferred_element_type=jnp.float32) + bih_ref[...]
    gh = jnp.dot(hv, whh_ref[...],
                 preferred_element_type=jnp.float32) + bhh_ref[...]
    r = jax.nn.sigmoid(gi[:, :HID] + gh[:, :HID])
    z = jax.nn.sigmoid(gi[:, HID:2 * HID] + gh[:, HID:2 * HID])
    ng = jnp.tanh(gi[:, 2 * HID:] + r * gh[:, 2 * HID:])
    hn = (1.0 - z) * ng + z * hv
    xnew = _ln(hn, g3_ref[...], b3_ref[...])

    xo_ref[...] = xnew
    ho_ref[...] = hn
    xno_ref[...] = jnp.dot(xnew, wn_ref[...], preferred_element_type=jnp.float32)
    aijo_ref[...] = jnp.dot(xnew, uij_ref[...], preferred_element_type=jnp.float32)


def _dense(agg2, asum4, x, h, wts):
    full = lambda shape: pl.BlockSpec(shape, lambda i: tuple(0 for _ in shape))
    row = lambda w: pl.BlockSpec((BLK, w), lambda i: (i, 0))
    in_specs = [
        pl.BlockSpec((NC, BLK, HID), lambda i: (0, i, 0)),
        pl.BlockSpec((NC, BLK, HEADS), lambda i: (0, i, 0)),
        row(HID), row(HID),
        full((HID, HID)), full((1, HID)), full((HID, HID)), full((1, HID)),
        full((1, HID)), full((1, HID)),
        full((HID, 4 * HID)), full((1, 4 * HID)),
        full((4 * HID, HID)), full((1, HID)), full((1, HID)), full((1, HID)),
        full((HID, 3 * HID)), full((HID, 3 * HID)),
        full((1, 3 * HID)), full((1, 3 * HID)),
        full((1, HID)), full((1, HID)),
        full((HID, HID)), full((HID, 2 * HEADS)),
    ]
    return pl.pallas_call(
        _dense_body,
        grid=(N // BLK,),
        in_specs=in_specs,
        out_specs=[row(HID), row(HID), row(HID), row(2 * HEADS)],
        out_shape=[jax.ShapeDtypeStruct((N, HID), jnp.float32),
                   jax.ShapeDtypeStruct((N, HID), jnp.float32),
                   jax.ShapeDtypeStruct((N, HID), jnp.float32),
                   jax.ShapeDtypeStruct((N, 2 * HEADS), jnp.float32)],
    )(agg2, asum4, x, h, *wts)


def _sc_weights_body(src_hbm, dst_hbm, ae_hbm, aij_hbm, iota_hbm, zeros_hbm,
                     w_hbm, asum_hbm, aij_v, s0, s1, d0, d1, a0, a1, w0, w1,
                     iota_v, asum_loc, asum_sp, si0, si1, so0, so1):
    c = lax.axis_index("c")
    s = lax.axis_index("s")
    srci = (s0, s1)
    dsti = (d0, d1)
    aev = (a0, a1)
    wv_ = (w0, w1)
    sidx = (si0, si1)
    sout = (so0, so1)

    pltpu.sync_copy(aij_hbm, aij_v)
    pltpu.sync_copy(iota_hbm, iota_v)
    pltpu.sync_copy(zeros_hbm.at[pl.ds(0, ASR)], asum_loc)

    @pl.when(s < 10)
    def _():
        pltpu.sync_copy(zeros_hbm.at[pl.ds(0, 32)],
                        asum_sp.at[pl.ds(s * 32, 32)])

    plsc.subcore_barrier()

    base = (c * NS + s) * EPT

    def fire_idx(i, u):
        off = base + i * CHUNK
        pltpu.async_copy(src_hbm.at[pl.ds(off, CHUNK)], srci[u], sidx[u])
        pltpu.async_copy(dst_hbm.at[pl.ds(off, CHUNK)], dsti[u], sidx[u])
        pltpu.async_copy(ae_hbm.at[pl.ds(off * HEADS, CHUNK * HEADS)],
                         aev[u], sidx[u])

    def wait_idx(u):
        pltpu.make_async_copy(src_hbm.at[pl.ds(0, CHUNK)], srci[u],
                              sidx[u]).wait()
        pltpu.make_async_copy(dst_hbm.at[pl.ds(0, CHUNK)], dsti[u],
                              sidx[u]).wait()
        pltpu.make_async_copy(ae_hbm.at[pl.ds(0, CHUNK * HEADS)], aev[u],
                              sidx[u]).wait()

    def fire_out(i, u):
        off = base + i * CHUNK
        return pltpu.async_copy(wv_[u], w_hbm.at[pl.ds(off * HEADS,
                                                       CHUNK * HEADS)],
                                sout[u])

    def wait_out(u):
        pltpu.make_async_copy(wv_[u], w_hbm.at[pl.ds(0, CHUNK * HEADS)],
                              sout[u]).wait()

    def compute(u):
        for g in range(CHUNK // 16):
            rid4 = (lax.iota(jnp.int32, 16) + g * 16) * HEADS
            dstg = dsti[u][pl.ds(g * 16, 16)]
            srcg = srci[u][pl.ds(g * 16, 16)]
            d8 = dstg * (2 * HEADS)
            s8 = srcg * (2 * HEADS) + HEADS
            d4 = dstg * HEADS
            for hh in range(HEADS):
                vi = plsc.load_gather(aij_v, [d8 + hh])
                vj = plsc.load_gather(aij_v, [s8 + hh])
                ve = plsc.load_gather(aev[u], [rid4 + hh])
                sv = vi + vj + ve
                sv = jnp.maximum(sv, 0.2 * sv)
                wvv = jnp.exp(sv)
                plsc.store_scatter(wv_[u], [rid4 + hh], wvv)
                f = d4 + hh
                plsc.addupdate_scatter(
                    asum_loc,
                    [lax.shift_right_logical(f, 7), lax.bitwise_and(f, 127)],
                    wvv)

    def sub(i, u, first=False, pf=True):
        wait_idx(u)
        if pf:
            fire_idx(i + 1, (u + 1) % 2)
        if not first:
            wait_out(u)
        compute(u)
        return fire_out(i, u)

    fire_idx(0, 0)
    sub(0, 0, first=True)
    sub(1, 1, first=True)

    @pl.loop(1, NCHUNK // 2 - 1)
    def _(j):
        sub(2 * j, 0)
        sub(2 * j + 1, 1)

    hs = [sub(NCHUNK - 2, 0), sub(NCHUNK - 1, 1, pf=False)]
    for h in hs:
        h.wait()

    for j in range(HEADS):
        pltpu.sync_copy(asum_loc.at[pl.ds(j * (ASR // HEADS), ASR // HEADS)],
                        asum_sp.at[iota_v.at[j]], add=True)

    plsc.subcore_barrier()

    @pl.when(s < 10)
    def _():
        pltpu.sync_copy(asum_sp.at[pl.ds(s * 32, 32)],
                        asum_hbm.at[c, pl.ds(s * 32, 32)])


CB = 64
NSLOT = 5
EPT0 = 7040
EPT1 = 13440
NCB0 = EPT0 // CB
NCB1 = EPT1 // CB


def _sc_aggr_body(src_hbm, dst_hbm, w_hbm, xn_hbm, zeros_hbm,
                  agg_hbm, *scr):
    srci = scr[0:NSLOT]
    dsti = scr[NSLOT:2 * NSLOT]
    wv = scr[2 * NSLOT:3 * NSLOT]
    rows = scr[3 * NSLOT:4 * NSLOT]
    acc_sp = scr[4 * NSLOT]
    sidx = scr[4 * NSLOT + 1:4 * NSLOT + 1 + NSLOT]
    sgat = scr[4 * NSLOT + 1 + NSLOT:4 * NSLOT + 1 + 2 * NSLOT]
    ssc = scr[4 * NSLOT + 1 + 2 * NSLOT:4 * NSLOT + 1 + 3 * NSLOT]

    c = lax.axis_index("c")
    s = lax.axis_index("s")

    pltpu.sync_copy(zeros_hbm.at[pl.ds(0, RPT)], acc_sp.at[pl.ds(s * RPT, RPT)])
    plsc.subcore_barrier()

    def wait_idx(u):
        pltpu.make_async_copy(src_hbm.at[pl.ds(0, CB)], srci[u], sidx[u]).wait()
        pltpu.make_async_copy(dst_hbm.at[pl.ds(0, CB)], dsti[u], sidx[u]).wait()
        pltpu.make_async_copy(w_hbm.at[pl.ds(0, CB * HEADS)], wv[u],
                              sidx[u]).wait()

    def fire_gather(u):
        pltpu.async_copy(xn_hbm.at[srci[u]], rows[u], sgat[u])

    def wait_gather(u):
        pltpu.make_async_copy(xn_hbm.at[pl.ds(0, CB)], rows[u], sgat[u]).wait()

    def fire_scatter(u):
        return pltpu.async_copy(rows[u], acc_sp.at[dsti[u]], ssc[u], add=True)

    def wait_scatter(u):
        pltpu.make_async_copy(xn_hbm.at[pl.ds(0, CB)], rows[u], ssc[u]).wait()

    def scale(u):
        ru = rows[u]
        wu = wv[u]

        @pl.loop(0, CB)
        def _(k):
            k4 = k * HEADS
            for hh in range(HEADS):
                ws = plsc.load_gather(wu, [jnp.broadcast_to(k4 + hh, (16,))])
                for q in range(HD // 16):
                    colo = hh * HD + q * 16
                    ru[k, pl.ds(colo, 16)] = ru[k, pl.ds(colo, 16)] * ws

    def run(ncb, base_e):
        def fire_idx(i, u):
            e = base_e + i * CB
            pltpu.async_copy(src_hbm.at[pl.ds(e, CB)], srci[u], sidx[u])
            pltpu.async_copy(dst_hbm.at[pl.ds(e, CB)], dsti[u], sidx[u])
            pltpu.async_copy(w_hbm.at[pl.ds(e * HEADS, CB * HEADS)], wv[u],
                             sidx[u])

        def sub(i, u, skip_ssc=False, pf_idx=True, pf_gat=True):
            wait_gather(u)
            scale(u)
            h = fire_scatter(u)
            if pf_idx:
                v3 = (u + 3) % NSLOT
                if not skip_ssc:
                    wait_scatter(v3)
                fire_idx(i + 3, v3)
            if pf_gat:
                v2 = (u + 2) % NSLOT
                wait_idx(v2)
                fire_gather(v2)
            return h

        fire_idx(0, 0)
        fire_idx(1, 1)
        fire_idx(2, 2)
        wait_idx(0)
        fire_gather(0)
        wait_idx(1)
        fire_gather(1)
        sub(0, 0, skip_ssc=True)
        sub(1, 1, skip_ssc=True)
        sub(2, 2)
        sub(3, 3)
        sub(4, 4)

        @pl.loop(1, ncb // NSLOT - 1)
        def _(j):
            i0 = j * NSLOT
            for u in range(NSLOT):
                sub(i0 + u, u)

        i0 = ncb - NSLOT
        hs = [sub(i0, 0),
              sub(i0 + 1, 1),
              sub(i0 + 2, 2, pf_idx=False),
              sub(i0 + 3, 3, pf_idx=False, pf_gat=False),
              sub(i0 + 4, 4, pf_idx=False, pf_gat=False)]
        for h in hs:
            h.wait()

    @pl.when(c == 0)
    def _():
        run(NCB0, s * EPT0)

    @pl.when(c == 1)
    def _():
        run(NCB1, NS * EPT0 + s * EPT1)

    plsc.subcore_barrier()
    pltpu.sync_copy(acc_sp.at[pl.ds(s * RPT, RPT)],
                    agg_hbm.at[c, pl.ds(s * RPT, RPT)])


def _sc_compiler_params():
    cp = pltpu.CompilerParams()
    if "needs_layout_passes" in pltpu.CompilerParams.__dataclass_fields__:
        cp = dataclasses.replace(cp, needs_layout_passes=False)
    return cp


@functools.cache
def _sc_weights_kernel():
    mesh = plsc.VectorSubcoreMesh(core_axis_name="c", subcore_axis_name="s")
    return pl.kernel(
        _sc_weights_body,
        out_type=[jax.ShapeDtypeStruct((E_PAD * HEADS,), jnp.float32),
                  jax.ShapeDtypeStruct((NC, ASR, HID), jnp.float32)],
        mesh=mesh,
        compiler_params=_sc_compiler_params(),
        scratch_types=(
            [pltpu.VMEM((N * 2 * HEADS,), jnp.float32)]
            + [pltpu.VMEM((CHUNK,), jnp.int32)] * 2
            + [pltpu.VMEM((CHUNK,), jnp.int32)] * 2
            + [pltpu.VMEM((CHUNK * HEADS,), jnp.float32)] * 2
            + [pltpu.VMEM((CHUNK * HEADS,), jnp.float32)] * 2
            + [pltpu.VMEM((HEADS, ASR // HEADS), jnp.int32)]
            + [pltpu.VMEM((ASR, HID), jnp.float32)]
            + [pltpu.VMEM_SHARED((ASR, HID), jnp.float32)]
            + [pltpu.SemaphoreType.DMA] * 4
        ),
    )


@functools.cache
def _sc_aggr_kernel():
    mesh = plsc.VectorSubcoreMesh(core_axis_name="c", subcore_axis_name="s")
    return pl.kernel(
        _sc_aggr_body,
        out_type=jax.ShapeDtypeStruct((NC, N_PAD, HID), jnp.float32),
        mesh=mesh,
        compiler_params=_sc_compiler_params(),
        scratch_types=(
            [pltpu.VMEM((CB,), jnp.int32)] * NSLOT
            + [pltpu.VMEM((CB,), jnp.int32)] * NSLOT
            + [pltpu.VMEM((CB * HEADS,), jnp.float32)] * NSLOT
            + [pltpu.VMEM((CB, HID), jnp.float32)] * NSLOT
            + [pltpu.VMEM_SHARED((N_PAD, HID), jnp.float32)]
            + [pltpu.SemaphoreType.DMA] * (3 * NSLOT)
        ),
    )


def kernel(x, edge_index, edge_attr, W_node, W_edge, att_w, W_scale, b_scale,
           W_ao, b_ao, g_ln1, b_ln1, W_int, b_int, W_out, b_out, g_ln2, b_ln2,
           W_ih, W_hh, b_ih, b_hh, g_ln3, b_ln3):
    src = edge_index[0]
    dst = edge_index[1]

    aw = att_w.reshape(HEADS, 3, HD)
    Wn4 = W_node.reshape(HID, HEADS, HD)
    We4 = W_edge.reshape(HID, HEADS, HD)
    U_i = jnp.einsum('khd,hd->kh', Wn4, aw[:, 0, :])
    U_e = jnp.einsum('khd,hd->kh', We4, aw[:, 1, :])
    U_j = jnp.einsum('khd,hd->kh', Wn4, aw[:, 2, :])
    U_ij = jnp.concatenate([U_i, U_j], axis=1)

    ae = _edge_logits(edge_attr, U_e)
    ae_pad = jnp.concatenate(
        [ae, jnp.full((E_PAD - E, HEADS), _NEG, jnp.float32)]).reshape(-1)
    src_pad = jnp.concatenate([src, jnp.zeros((E_PAD - E,), jnp.int32)])
    dst_pad = jnp.concatenate([dst, jnp.zeros((E_PAD - E,), jnp.int32)])
    zeros = jnp.zeros((N_PAD, HID), jnp.float32)
    iota = jnp.arange(ASR, dtype=jnp.int32).reshape(HEADS, ASR // HEADS)

    r2 = lambda v: v.reshape(1, -1)
    wts = (W_scale, r2(b_scale), W_ao, r2(b_ao), r2(g_ln1), r2(b_ln1),
           W_int, r2(b_int), W_out, r2(b_out), r2(g_ln2), r2(b_ln2),
           W_ih.T, W_hh.T, r2(b_ih), r2(b_hh), r2(g_ln3), r2(b_ln3),
           W_node, U_ij)

    xn, aij = _node_proj(x, W_node, U_ij)
    h = x
    for _ in range(T):
        w_e, asum2 = _sc_weights_kernel()(src_pad, dst_pad, ae_pad,
                                          aij.reshape(-1), iota, zeros)
        agg2 = _sc_aggr_kernel()(src_pad, dst_pad, w_e, xn, zeros)
        asum4 = asum2.reshape(NC, N_PAD, HEADS)
        x, h, xn, aij = _dense(agg2, asum4, x, h, wts)
    return x


# speedup vs baseline: 7.1428x; 1.1713x over previous
import dataclasses
import functools

import jax
import jax.numpy as jnp
from jax import lax
from jax.experimental import pallas as pl
from jax.experimental.pallas import tpu as pltpu
from jax.experimental.pallas import tpu_sc as plsc

HID = 128
HEADS = 4
HD = HID // HEADS
T = 3
N = 10000
E = 320000

NC = 2
NS = 16
NW = NC * NS
CHUNK = 128
E_PAD = 327680
EPT = E_PAD // NW
NCHUNK = EPT // CHUNK
N_PAD = 10240
RPT = N_PAD // NS
ASR = N_PAD * HEADS // HID

BLK = 400
BLKE = 1000

_NEG = -1e30


def _ln(v, g, b):
    u = jnp.mean(v, axis=-1, keepdims=True)
    d = v - u
    var = jnp.mean(d * d, axis=-1, keepdims=True)
    return d / jnp.sqrt(var + 1e-12) * g + b


def _edge_logits_body(ea_ref, ue_ref, out_ref):
    out_ref[...] = jnp.dot(ea_ref[...], ue_ref[...],
                           preferred_element_type=jnp.float32)


def _edge_logits(edge_attr, U_e):
    return pl.pallas_call(
        _edge_logits_body,
        grid=(E // BLKE,),
        in_specs=[pl.BlockSpec((BLKE, HID), lambda i: (i, 0)),
                  pl.BlockSpec((HID, HEADS), lambda i: (0, 0))],
        out_specs=pl.BlockSpec((BLKE, HEADS), lambda i: (i, 0)),
        out_shape=jax.ShapeDtypeStruct((E, HEADS), jnp.float32),
    )(edge_attr, U_e)


def _node_proj_body(x_ref, wn_ref, uij_ref, xn_ref, aij_ref):
    xv = x_ref[...]
    xn_ref[...] = jnp.dot(xv, wn_ref[...], preferred_element_type=jnp.float32)
    aij_ref[...] = jnp.dot(xv, uij_ref[...], preferred_element_type=jnp.float32)


def _node_proj(x, W_node, U_ij):
    return pl.pallas_call(
        _node_proj_body,
        grid=(N // BLK,),
        in_specs=[pl.BlockSpec((BLK, HID), lambda i: (i, 0)),
                  pl.BlockSpec((HID, HID), lambda i: (0, 0)),
                  pl.BlockSpec((HID, 2 * HEADS), lambda i: (0, 0))],
        out_specs=[pl.BlockSpec((BLK, HID), lambda i: (i, 0)),
                   pl.BlockSpec((BLK, 2 * HEADS), lambda i: (i, 0))],
        out_shape=[jax.ShapeDtypeStruct((N, HID), jnp.float32),
                   jax.ShapeDtypeStruct((N, 2 * HEADS), jnp.float32)],
    )(x, W_node, U_ij)


def _dense_body(agg_ref, asum_ref, x_ref, h_ref,
                ws_ref, bs_ref, wao_ref, bao_ref, g1_ref, b1_ref,
                wi_ref, bi_ref, wo_ref, bo_ref, g2_ref, b2_ref,
                wih_ref, whh_ref, bih_ref, bhh_ref, g3_ref, b3_ref,
                wn_ref, uij_ref,
                xo_ref, ho_ref, xno_ref, aijo_ref):
    aggr = agg_ref[0] + agg_ref[1]
    asum = asum_ref[0] + asum_ref[1]
    recip = 1.0 / (asum + 1e-16)
    lane_head = lax.broadcasted_iota(jnp.int32, (HEADS, HID), 1) // HD
    head_id = lax.broadcasted_iota(jnp.int32, (HEADS, HID), 0)
    sel = (lane_head == head_id).astype(jnp.float32)
    attn_in = aggr * jnp.dot(recip, sel, preferred_element_type=jnp.float32)

    xv = x_ref[...]
    attn = jnp.dot(attn_in, ws_ref[...],
                   preferred_element_type=jnp.float32) + bs_ref[...]
    ao = jnp.dot(attn, wao_ref[...],
                 preferred_element_type=jnp.float32) + bao_ref[...]
    ao = _ln(ao + xv, g1_ref[...], b1_ref[...])

    inter = jnp.dot(ao, wi_ref[...],
                    preferred_element_type=jnp.float32) + bi_ref[...]
    inter = 0.5 * inter * (1.0 + lax.erf(inter * 0.7071067811865476))
    m = jnp.dot(inter, wo_ref[...],
                preferred_element_type=jnp.float32) + bo_ref[...]
    m = _ln(m + ao, g2_ref[...], b2_ref[...])

    hv = h_ref[...]
    gi = jnp.dot(m, wih_ref[...],
                 preferred_element_type=jnp.float32) + bih_ref[...]
    gh = jnp.dot(hv, whh_ref[...],
                 preferred_element_type=jnp.float32) + bhh_ref[...]
    r = jax.nn.sigmoid(gi[:, :HID] + gh[:, :HID])
    z = jax.nn.sigmoid(gi[:, HID:2 * HID] + gh[:, HID:2 * HID])
    ng = jnp.tanh(gi[:, 2 * HID:] + r * gh[:, 2 * HID:])
    hn = (1.0 - z) * ng + z * hv
    xnew = _ln(hn, g3_ref[...], b3_ref[...])

    xo_ref[...] = xnew
    ho_ref[...] = hn
    xno_ref[...] = jnp.dot(xnew, wn_ref[...], preferred_element_type=jnp.float32)
    aijo_ref[...] = jnp.dot(xnew, uij_ref[...], preferred_element_type=jnp.float32)


def _dense(agg2, asum4, x, h, wts):
    full = lambda shape: pl.BlockSpec(shape, lambda i: tuple(0 for _ in shape))
    row = lambda w: pl.BlockSpec((BLK, w), lambda i: (i, 0))
    in_specs = [
        pl.BlockSpec((NC, BLK, HID), lambda i: (0, i, 0)),
        pl.BlockSpec((NC, BLK, HEADS), lambda i: (0, i, 0)),
        row(HID), row(HID),
        full((HID, HID)), full((1, HID)), full((HID, HID)), full((1, HID)),
        full((1, HID)), full((1, HID)),
        full((HID, 4 * HID)), full((1, 4 * HID)),
        full((4 * HID, HID)), full((1, HID)), full((1, HID)), full((1, HID)),
        full((HID, 3 * HID)), full((HID, 3 * HID)),
        full((1, 3 * HID)), full((1, 3 * HID)),
        full((1, HID)), full((1, HID)),
        full((HID, HID)), full((HID, 2 * HEADS)),
    ]
    return pl.pallas_call(
        _dense_body,
        grid=(N // BLK,),
        in_specs=in_specs,
        out_specs=[row(HID), row(HID), row(HID), row(2 * HEADS)],
        out_shape=[jax.ShapeDtypeStruct((N, HID), jnp.float32),
                   jax.ShapeDtypeStruct((N, HID), jnp.float32),
                   jax.ShapeDtypeStruct((N, HID), jnp.float32),
                   jax.ShapeDtypeStruct((N, 2 * HEADS), jnp.float32)],
    )(agg2, asum4, x, h, *wts)


def _sc_weights_body(src_hbm, dst_hbm, ae_hbm, aij_hbm, iota_hbm, zeros_hbm,
                     w_hbm, asum_hbm, aij_v, s0, s1, d0, d1, a0, a1, w0, w1,
                     iota_v, asum_loc, asum_sp, si0, si1, so0, so1):
    c = lax.axis_index("c")
    s = lax.axis_index("s")
    srci = (s0, s1)
    dsti = (d0, d1)
    aev = (a0, a1)
    wv_ = (w0, w1)
    sidx = (si0, si1)
    sout = (so0, so1)

    pltpu.sync_copy(aij_hbm, aij_v)
    pltpu.sync_copy(iota_hbm, iota_v)
    pltpu.sync_copy(zeros_hbm.at[pl.ds(0, ASR)], asum_loc)

    @pl.when(s < 10)
    def _():
        pltpu.sync_copy(zeros_hbm.at[pl.ds(0, 32)],
                        asum_sp.at[pl.ds(s * 32, 32)])

    plsc.subcore_barrier()

    base = (c * NS + s) * EPT

    def fire_idx(i, u):
        off = base + i * CHUNK
        pltpu.async_copy(src_hbm.at[pl.ds(off, CHUNK)], srci[u], sidx[u])
        pltpu.async_copy(dst_hbm.at[pl.ds(off, CHUNK)], dsti[u], sidx[u])
        pltpu.async_copy(ae_hbm.at[pl.ds(off * HEADS, CHUNK * HEADS)],
                         aev[u], sidx[u])

    def wait_idx(u):
        pltpu.make_async_copy(src_hbm.at[pl.ds(0, CHUNK)], srci[u],
                              sidx[u]).wait()
        pltpu.make_async_copy(dst_hbm.at[pl.ds(0, CHUNK)], dsti[u],
                              sidx[u]).wait()
        pltpu.make_async_copy(ae_hbm.at[pl.ds(0, CHUNK * HEADS)], aev[u],
                              sidx[u]).wait()

    def fire_out(i, u):
        off = base + i * CHUNK
        return pltpu.async_copy(wv_[u], w_hbm.at[pl.ds(off * HEADS,
                                                       CHUNK * HEADS)],
                                sout[u])

    def wait_out(u):
        pltpu.make_async_copy(wv_[u], w_hbm.at[pl.ds(0, CHUNK * HEADS)],
                              sout[u]).wait()

    def compute(u):
        for g in range(CHUNK // 16):
            rid4 = (lax.iota(jnp.int32, 16) + g * 16) * HEADS
            dstg = dsti[u][pl.ds(g * 16, 16)]
            srcg = srci[u][pl.ds(g * 16, 16)]
            d8 = dstg * (2 * HEADS)
            s8 = srcg * (2 * HEADS) + HEADS
            d4 = dstg * HEADS
            for hh in range(HEADS):
                vi = plsc.load_gather(aij_v, [d8 + hh])
                vj = plsc.load_gather(aij_v, [s8 + hh])
                ve = plsc.load_gather(aev[u], [rid4 + hh])
                sv = vi + vj + ve
                sv = jnp.maximum(sv, 0.2 * sv)
                wvv = jnp.exp(sv)
                plsc.store_scatter(wv_[u], [rid4 + hh], wvv)
                f = d4 + hh
                plsc.addupdate_scatter(
                    asum_loc,
                    [lax.shift_right_logical(f, 7), lax.bitwise_and(f, 127)],
                    wvv)

    def sub(i, u, first=False, pf=True):
        wait_idx(u)
        if pf:
            fire_idx(i + 1, (u + 1) % 2)
        if not first:
            wait_out(u)
        compute(u)
        return fire_out(i, u)

    fire_idx(0, 0)
    sub(0, 0, first=True)
    sub(1, 1, first=True)

    @pl.loop(1, NCHUNK // 2 - 1)
    def _(j):
        sub(2 * j, 0)
        sub(2 * j + 1, 1)

    hs = [sub(NCHUNK - 2, 0), sub(NCHUNK - 1, 1, pf=False)]
    for h in hs:
        h.wait()

    for j in range(HEADS):
        pltpu.sync_copy(asum_loc.at[pl.ds(j * (ASR // HEADS), ASR // HEADS)],
                        asum_sp.at[iota_v.at[j]], add=True)

    plsc.subcore_barrier()

    @pl.when(s < 10)
    def _():
        pltpu.sync_copy(asum_sp.at[pl.ds(s * 32, 32)],
                        asum_hbm.at[c, pl.ds(s * 32, 32)])


CB = 64
NSLOT = 5
EPT0 = 13440
EPT1 = 7040
NCB0 = EPT0 // CB
NCB1 = EPT1 // CB


def _sc_aggr_body(src_hbm, dst_hbm, w_hbm, xn_hbm, zeros_hbm,
                  agg_hbm, *scr):
    srci = scr[0:NSLOT]
    dsti = scr[NSLOT:2 * NSLOT]
    wv = scr[2 * NSLOT:3 * NSLOT]
    rows = scr[3 * NSLOT:4 * NSLOT]
    acc_sp = scr[4 * NSLOT]
    sidx = scr[4 * NSLOT + 1:4 * NSLOT + 1 + NSLOT]
    sgat = scr[4 * NSLOT + 1 + NSLOT:4 * NSLOT + 1 + 2 * NSLOT]
    ssc = scr[4 * NSLOT + 1 + 2 * NSLOT:4 * NSLOT + 1 + 3 * NSLOT]

    c = lax.axis_index("c")
    s = lax.axis_index("s")

    pltpu.sync_copy(zeros_hbm.at[pl.ds(0, RPT)], acc_sp.at[pl.ds(s * RPT, RPT)])
    plsc.subcore_barrier()

    def wait_idx(u):
        pltpu.make_async_copy(src_hbm.at[pl.ds(0, CB)], srci[u], sidx[u]).wait()
        pltpu.make_async_copy(dst_hbm.at[pl.ds(0, CB)], dsti[u], sidx[u]).wait()
        pltpu.make_async_copy(w_hbm.at[pl.ds(0, CB * HEADS)], wv[u],
                              sidx[u]).wait()

    def fire_gather(u):
        pltpu.async_copy(xn_hbm.at[srci[u]], rows[u], sgat[u])

    def wait_gather(u):
        pltpu.make_async_copy(xn_hbm.at[pl.ds(0, CB)], rows[u], sgat[u]).wait()

    def fire_scatter(u):
        return pltpu.async_copy(rows[u], acc_sp.at[dsti[u]], ssc[u], add=True)

    def wait_scatter(u):
        pltpu.make_async_copy(xn_hbm.at[pl.ds(0, CB)], rows[u], ssc[u]).wait()

    def scale(u):
        ru = rows[u]
        wu = wv[u]

        @pl.loop(0, CB)
        def _(k):
            k4 = k * HEADS
            for hh in range(HEADS):
                ws = plsc.load_gather(wu, [jnp.broadcast_to(k4 + hh, (16,))])
                for q in range(HD // 16):
                    colo = hh * HD + q * 16
                    ru[k, pl.ds(colo, 16)] = ru[k, pl.ds(colo, 16)] * ws

    def run(ncb, base_e):
        def fire_idx(i, u):
            e = base_e + i * CB
            pltpu.async_copy(src_hbm.at[pl.ds(e, CB)], srci[u], sidx[u])
            pltpu.async_copy(dst_hbm.at[pl.ds(e, CB)], dsti[u], sidx[u])
            pltpu.async_copy(w_hbm.at[pl.ds(e * HEADS, CB * HEADS)], wv[u],
                             sidx[u])

        def sub(i, u, skip_ssc=False, pf_idx=True, pf_gat=True):
            wait_gather(u)
            scale(u)
            h = fire_scatter(u)
            if pf_idx:
                v3 = (u + 3) % NSLOT
                if not skip_ssc:
                    wait_scatter(v3)
                fire_idx(i + 3, v3)
            if pf_gat:
                v2 = (u + 2) % NSLOT
                wait_idx(v2)
                fire_gather(v2)
            return h

        fire_idx(0, 0)
        fire_idx(1, 1)
        fire_idx(2, 2)
        wait_idx(0)
        fire_gather(0)
        wait_idx(1)
        fire_gather(1)
        sub(0, 0, skip_ssc=True)
        sub(1, 1, skip_ssc=True)
        sub(2, 2)
        sub(3, 3)
        sub(4, 4)

        @pl.loop(1, ncb // NSLOT - 1)
        def _(j):
            i0 = j * NSLOT
            for u in range(NSLOT):
                sub(i0 + u, u)

        i0 = ncb - NSLOT
        hs = [sub(i0, 0),
              sub(i0 + 1, 1),
              sub(i0 + 2, 2, pf_idx=False),
              sub(i0 + 3, 3, pf_idx=False, pf_gat=False),
              sub(i0 + 4, 4, pf_idx=False, pf_gat=False)]
        for h in hs:
            h.wait()

    @pl.when(c == 0)
    def _():
        run(NCB0, s * EPT0)

    @pl.when(c == 1)
    def _():
        run(NCB1, NS * EPT0 + s * EPT1)

    plsc.subcore_barrier()
    pltpu.sync_copy(acc_sp.at[pl.ds(s * RPT, RPT)],
                    agg_hbm.at[c, pl.ds(s * RPT, RPT)])


def _sc_compiler_params():
    cp = pltpu.CompilerParams()
    if "needs_layout_passes" in pltpu.CompilerParams.__dataclass_fields__:
        cp = dataclasses.replace(cp, needs_layout_passes=False)
    return cp


@functools.cache
def _sc_weights_kernel():
    mesh = plsc.VectorSubcoreMesh(core_axis_name="c", subcore_axis_name="s")
    return pl.kernel(
        _sc_weights_body,
        out_type=[jax.ShapeDtypeStruct((E_PAD * HEADS,), jnp.float32),
                  jax.ShapeDtypeStruct((NC, ASR, HID), jnp.float32)],
        mesh=mesh,
        compiler_params=_sc_compiler_params(),
        scratch_types=(
            [pltpu.VMEM((N * 2 * HEADS,), jnp.float32)]
            + [pltpu.VMEM((CHUNK,), jnp.int32)] * 2
            + [pltpu.VMEM((CHUNK,), jnp.int32)] * 2
            + [pltpu.VMEM((CHUNK * HEADS,), jnp.float32)] * 2
            + [pltpu.VMEM((CHUNK * HEADS,), jnp.float32)] * 2
            + [pltpu.VMEM((HEADS, ASR // HEADS), jnp.int32)]
            + [pltpu.VMEM((ASR, HID), jnp.float32)]
            + [pltpu.VMEM_SHARED((ASR, HID), jnp.float32)]
            + [pltpu.SemaphoreType.DMA] * 4
        ),
    )


@functools.cache
def _sc_aggr_kernel():
    mesh = plsc.VectorSubcoreMesh(core_axis_name="c", subcore_axis_name="s")
    return pl.kernel(
        _sc_aggr_body,
        out_type=jax.ShapeDtypeStruct((NC, N_PAD, HID), jnp.float32),
        mesh=mesh,
        compiler_params=_sc_compiler_params(),
        scratch_types=(
            [pltpu.VMEM((CB,), jnp.int32)] * NSLOT
            + [pltpu.VMEM((CB,), jnp.int32)] * NSLOT
            + [pltpu.VMEM((CB * HEADS,), jnp.float32)] * NSLOT
            + [pltpu.VMEM((CB, HID), jnp.float32)] * NSLOT
            + [pltpu.VMEM_SHARED((N_PAD, HID), jnp.float32)]
            + [pltpu.SemaphoreType.DMA] * (3 * NSLOT)
        ),
    )


def kernel(x, edge_index, edge_attr, W_node, W_edge, att_w, W_scale, b_scale,
           W_ao, b_ao, g_ln1, b_ln1, W_int, b_int, W_out, b_out, g_ln2, b_ln2,
           W_ih, W_hh, b_ih, b_hh, g_ln3, b_ln3):
    src = edge_index[0]
    dst = edge_index[1]

    aw = att_w.reshape(HEADS, 3, HD)
    Wn4 = W_node.reshape(HID, HEADS, HD)
    We4 = W_edge.reshape(HID, HEADS, HD)
    U_i = jnp.einsum('khd,hd->kh', Wn4, aw[:, 0, :])
    U_e = jnp.einsum('khd,hd->kh', We4, aw[:, 1, :])
    U_j = jnp.einsum('khd,hd->kh', Wn4, aw[:, 2, :])
    U_ij = jnp.concatenate([U_i, U_j], axis=1)

    ae = _edge_logits(edge_attr, U_e)
    ae_pad = jnp.concatenate(
        [ae, jnp.full((E_PAD - E, HEADS), _NEG, jnp.float32)]).reshape(-1)
    src_pad = jnp.concatenate([src, jnp.zeros((E_PAD - E,), jnp.int32)])
    dst_pad = jnp.concatenate([dst, jnp.zeros((E_PAD - E,), jnp.int32)])
    zeros = jnp.zeros((N_PAD, HID), jnp.float32)
    iota = jnp.arange(ASR, dtype=jnp.int32).reshape(HEADS, ASR // HEADS)

    r2 = lambda v: v.reshape(1, -1)
    wts = (W_scale, r2(b_scale), W_ao, r2(b_ao), r2(g_ln1), r2(b_ln1),
           W_int, r2(b_int), W_out, r2(b_out), r2(g_ln2), r2(b_ln2),
           W_ih.T, W_hh.T, r2(b_ih), r2(b_hh), r2(g_ln3), r2(b_ln3),
           W_node, U_ij)

    xn, aij = _node_proj(x, W_node, U_ij)
    h = x
    for _ in range(T):
        w_e, asum2 = _sc_weights_kernel()(src_pad, dst_pad, ae_pad,
                                          aij.reshape(-1), iota, zeros)
        agg2 = _sc_aggr_kernel()(src_pad, dst_pad, w_e, xn, zeros)
        asum4 = asum2.reshape(NC, N_PAD, HEADS)
        x, h, xn, aij = _dense(agg2, asum4, x, h, wts)
    return x
